# baseline (speedup 1.0000x reference)
import functools

import jax
import jax.numpy as jnp
from jax import lax
from jax.experimental import pallas as pl
from jax.experimental.pallas import tpu as pltpu
from jax.experimental.pallas import tpu_sc as plsc

N = 10000
NPAD = 10240
E = 320000
D = 128
G = 64
H = 64
OUT = 10

NC = 2
NS = 16
NW = NC * NS
KB = 80
NB = 125
EPW = NB * KB
RPT = NPAD // NS

_mesh = plsc.VectorSubcoreMesh(core_axis_name="c", subcore_axis_name="s")


def _sc_deg_body(dst_hbm, out_hbm, acc_sh, dst_v, ones_v, zero_v, sem_i):
    c = lax.axis_index("c")
    s = lax.axis_index("s")
    wid = c * NS + s
    pltpu.async_copy(dst_hbm.at[wid], dst_v, sem_i).wait()
    for i in range(KB // 16):
        ones_v[pl.ds(i * 16, 16)] = jnp.ones((16,), jnp.float32)
    for i in range(8):
        zero_v[pl.ds(i * 16, 16)] = jnp.zeros((16,), jnp.float32)

    def zloop(j, carry):
        pltpu.sync_copy(zero_v, acc_sh.at[pl.ds(s * RPT + j * 128, 128)])
        return carry

    lax.fori_loop(0, RPT // 128, zloop, 0)
    plsc.subcore_barrier()
    adds = [pltpu.async_copy(ones_v, acc_sh.at[dst_v.at[j]], sem_i, add=True)
            for j in range(NB)]
    for a in adds:
        a.wait()
    plsc.subcore_barrier()
    pltpu.sync_copy(acc_sh.at[pl.ds(s * RPT, RPT)],
                    out_hbm.at[pl.ds(c * NPAD + s * RPT, RPT)])


_deg_call = pl.kernel(
    _sc_deg_body,
    mesh=_mesh,
    out_type=jax.ShapeDtypeStruct((NC * NPAD,), jnp.float32),
    scratch_types=[
        pltpu.VMEM_SHARED((NPAD,), jnp.float32),
        pltpu.VMEM((NB, KB), jnp.int32),
        pltpu.VMEM((KB,), jnp.float32),
        pltpu.VMEM((128,), jnp.float32),
        pltpu.SemaphoreType.DMA,
    ],
)


_ZR = 8
_RB4 = 4
_RI = 8


def _sc_edge_body(h2_hbm, src_hbm, dst_hbm, out_hbm, acc_sh,
                  src_v, dst_v, r0, r1, r2, r3, zrows_v,
                  g0, g1, g2, g3, t0, t1, t2, t3,
                  i0, i1, i2, i3, i4, i5, i6, i7, sem_z):
    c = lax.axis_index("c")
    s = lax.axis_index("s")
    wid = c * NS + s
    rows = (r0, r1, r2, r3)
    gsems = (g0, g1, g2, g3)
    ssems = (t0, t1, t2, t3)
    isems = (i0, i1, i2, i3, i4, i5, i6, i7)

    def idx_issue(j, q):
        pltpu.async_copy(src_hbm.at[wid, pl.ds(j, 1)], src_v.at[pl.ds(q, 1)],
                         isems[q])
        pltpu.async_copy(dst_hbm.at[wid, pl.ds(j, 1)], dst_v.at[pl.ds(q, 1)],
                         isems[q])

    def idx_wait(q):
        pltpu.make_async_copy(src_hbm.at[wid, pl.ds(0, 1)],
                              src_v.at[pl.ds(q, 1)], isems[q]).wait()
        pltpu.make_async_copy(dst_hbm.at[wid, pl.ds(0, 1)],
                              dst_v.at[pl.ds(q, 1)], isems[q]).wait()

    for q in range(_RI):
        idx_issue(q, q)

    for r in range(_ZR):
        for q in range(D // 16):
            zrows_v[r, pl.ds(q * 16, 16)] = jnp.zeros((16,), jnp.float32)
    zcopies = [
        pltpu.async_copy(zrows_v, acc_sh.at[pl.ds(s * RPT + j * _ZR, _ZR)],
                         sem_z)
        for j in range(RPT // _ZR)
    ]
    for zc in zcopies:
        zc.wait()
    plsc.subcore_barrier()

    for j in range(3):
        idx_wait(j)
        pltpu.async_copy(h2_hbm.at[src_v.at[j]], rows[j], gsems[j])

    def eloop(i, carry):
        for qq in range(_RI):
            j = _RI * i + qq

            @pl.when(j < NB)
            def _body():
                p = qq % _RB4
                p3 = (qq + 3) % _RB4
                q3 = (qq + 3) % _RI

                pltpu.make_async_copy(h2_hbm.at[src_v.at[qq]],
                                      rows[p], gsems[p]).wait()
                pltpu.async_copy(rows[p], acc_sh.at[dst_v.at[qq]], ssems[p],
                                 add=True)

                @pl.when(j + 3 < NB)
                def _regather():
                    idx_wait(q3)

                    @pl.when(j - 1 >= 0)
                    def _wait_prev_scatter():
                        pltpu.make_async_copy(
                            rows[p3], acc_sh.at[dst_v.at[q3]],
                            ssems[p3]).wait()

                        @pl.when(jnp.logical_and(j + 7 < NB, j + 7 >= _RI))
                        def _reidx():
                            idx_issue(j + 7, (qq + 7) % _RI)

                    pltpu.async_copy(h2_hbm.at[src_v.at[q3]], rows[p3],
                                     gsems[p3])
        return carry

    lax.fori_loop(0, (NB + _RI - 1) // _RI, eloop, 0)
    for k in range(_RB4):
        pltpu.make_async_copy(rows[k], acc_sh.at[dst_v.at[k]],
                              ssems[k]).wait()
    plsc.subcore_barrier()
    pltpu.sync_copy(acc_sh.at[pl.ds(s * RPT, RPT)],
                    out_hbm.at[pl.ds(c * NPAD + s * RPT, RPT)])


_edge_call = pl.kernel(
    _sc_edge_body,
    mesh=_mesh,
    out_type=jax.ShapeDtypeStruct((NC * NPAD, D), jnp.float32),
    scratch_types=[
        pltpu.VMEM_SHARED((NPAD, D), jnp.float32),
        pltpu.VMEM((_RI, KB), jnp.int32),
        pltpu.VMEM((_RI, KB), jnp.int32),
        pltpu.VMEM((KB, D), jnp.float32),
        pltpu.VMEM((KB, D), jnp.float32),
        pltpu.VMEM((KB, D), jnp.float32),
        pltpu.VMEM((KB, D), jnp.float32),
        pltpu.VMEM((_ZR, D), jnp.float32),
    ] + [pltpu.SemaphoreType.DMA] * 17,
)


def _dinv_body(indeg_ref, out_ref):
    blk = indeg_ref[...]
    dsum = 1.0 + blk[0] + blk[1]
    dinv = lax.rsqrt(dsum)
    ones = jnp.ones((1, 128), jnp.float32)
    for r in range(8):
        out_ref[pl.ds(r * 128, 128), :] = lax.dot_general(
            dinv[r:r + 1, :], ones, (((0,), (0,)), ((), ())),
            preferred_element_type=jnp.float32)


def _dinv_call(indeg):
    return pl.pallas_call(
        _dinv_body,
        grid=(NPAD // 1024,),
        in_specs=[pl.BlockSpec((2, 8, 128), lambda i: (0, i, 0))],
        out_specs=pl.BlockSpec((1024, 128), lambda i: (i, 0)),
        out_shape=jax.ShapeDtypeStruct((NPAD, D), jnp.float32),
    )(indeg)


_RB = 1000
_NRB = N // _RB


def _mm_scale_body(x_ref, w_ref, dinv_ref, o_ref):
    o_ref[...] = jnp.dot(x_ref[...], w_ref[...],
                         preferred_element_type=jnp.float32) * dinv_ref[...]


def _mm_scale(x, w, dinv_b):
    return pl.pallas_call(
        _mm_scale_body,
        grid=(_NRB,),
        in_specs=[
            pl.BlockSpec((_RB, D), lambda i: (i, 0)),
            pl.BlockSpec((D, D), lambda i: (0, 0)),
            pl.BlockSpec((_RB, D), lambda i: (i, 0)),
        ],
        out_specs=pl.BlockSpec((_RB, D), lambda i: (i, 0)),
        out_shape=jax.ShapeDtypeStruct((N, D), jnp.float32),
    )(x, w, dinv_b)


def _layer_body(acc_ref, h2_ref, dinv_ref, b_ref, w_ref, o_ref):
    a = acc_ref[...]
    dinv = dinv_ref[...]
    pre = dinv * (a[0] + a[1] + h2_ref[...]) + b_ref[...]
    xn = jnp.maximum(pre, 0.0)
    o_ref[...] = jnp.dot(xn, w_ref[...],
                         preferred_element_type=jnp.float32) * dinv


def _layer(acc, h2, dinv_b, b, w):
    return pl.pallas_call(
        _layer_body,
        grid=(_NRB,),
        in_specs=[
            pl.BlockSpec((2, _RB, D), lambda i: (0, i, 0)),
            pl.BlockSpec((_RB, D), lambda i: (i, 0)),
            pl.BlockSpec((_RB, D), lambda i: (i, 0)),
            pl.BlockSpec((1, D), lambda i: (0, 0)),
            pl.BlockSpec((D, D), lambda i: (0, 0)),
        ],
        out_specs=pl.BlockSpec((_RB, D), lambda i: (i, 0)),
        out_shape=jax.ShapeDtypeStruct((N, D), jnp.float32),
    )(acc, h2, dinv_b, b, w)


def _head_body(acc_ref, h2_ref, dinv_ref, b_ref, batch_ref,
               wm0_ref, bm0_ref, wm1_ref, bm1_ref,
               emb_ref, logits_ref, probs_ref,
               pooled_ref, counts_ref):
    i = pl.program_id(0)
    a = acc_ref[...]
    pre = dinv_ref[...] * (a[0] + a[1] + h2_ref[...]) + b_ref[...]
    emb = jnp.maximum(pre, 0.0)
    emb_ref[...] = emb

    bat = batch_ref[pl.ds(i, 1), :]
    gids = lax.broadcasted_iota(jnp.int32, (G, 1), 0)
    onehot = (bat == gids).astype(jnp.float32)

    @pl.when(i == 0)
    def _init():
        pooled_ref[...] = jnp.zeros((G, D), jnp.float32)
        counts_ref[...] = jnp.zeros((G, D), jnp.float32)

    pooled_ref[...] += jnp.dot(onehot, emb, preferred_element_type=jnp.float32)
    cnt = jnp.sum(onehot, axis=1, keepdims=True)
    counts_ref[...] += jnp.broadcast_to(cnt, (G, D))

    @pl.when(i == _NRB - 1)
    def _final():
        pooled = pooled_ref[...] / jnp.maximum(counts_ref[...], 1.0)
        z = jnp.dot(pooled, wm0_ref[...],
                    preferred_element_type=jnp.float32) + bm0_ref[...]
        z = jnp.where(z > 0.0, z, jnp.exp(jnp.minimum(z, 0.0)) - 1.0)
        logits = jnp.dot(z, wm1_ref[...],
                         preferred_element_type=jnp.float32) + bm1_ref[...]
        logits_ref[...] = logits
        m = jnp.max(logits, axis=-1, keepdims=True)
        e = jnp.exp(logits - m)
        probs_ref[...] = e / jnp.sum(e, axis=-1, keepdims=True)


def _head(acc, h2, dinv_b, b, batch2d, wm0, bm0, wm1, bm1):
    return pl.pallas_call(
        _head_body,
        grid=(_NRB,),
        in_specs=[
            pl.BlockSpec((2, _RB, D), lambda i: (0, i, 0)),
            pl.BlockSpec((_RB, D), lambda i: (i, 0)),
            pl.BlockSpec((_RB, D), lambda i: (i, 0)),
            pl.BlockSpec((1, D), lambda i: (0, 0)),
            pl.BlockSpec((_NRB, _RB), lambda i: (0, 0)),
            pl.BlockSpec((D, H), lambda i: (0, 0)),
            pl.BlockSpec((1, H), lambda i: (0, 0)),
            pl.BlockSpec((H, OUT), lambda i: (0, 0)),
            pl.BlockSpec((1, OUT), lambda i: (0, 0)),
        ],
        out_specs=[
            pl.BlockSpec((_RB, D), lambda i: (i, 0)),
            pl.BlockSpec((G, OUT), lambda i: (0, 0)),
            pl.BlockSpec((G, OUT), lambda i: (0, 0)),
        ],
        out_shape=[
            jax.ShapeDtypeStruct((N, D), jnp.float32),
            jax.ShapeDtypeStruct((G, OUT), jnp.float32),
            jax.ShapeDtypeStruct((G, OUT), jnp.float32),
        ],
        scratch_shapes=[
            pltpu.VMEM((G, D), jnp.float32),
            pltpu.VMEM((G, D), jnp.float32),
        ],
    )(acc, h2, dinv_b, b, batch2d, wm0, bm0, wm1, bm1)


def kernel(x, edge_index, batch, W1, b1, W2, b2, W3, b3, Wm0, bm0, Wm1, bm1):
    src = edge_index[0].reshape(NW, NB, KB)
    dst = edge_index[1].reshape(NW, NB, KB)

    indeg = _deg_call(dst).reshape(NC, NPAD // 128, 128)
    dinv_n = _dinv_call(indeg)

    h2 = _mm_scale(x, W1, dinv_n)
    acc = _edge_call(h2, src, dst).reshape(NC, NPAD, D)
    h2 = _layer(acc, h2, dinv_n, b1.reshape(1, D), W2)
    acc = _edge_call(h2, src, dst).reshape(NC, NPAD, D)
    h2 = _layer(acc, h2, dinv_n, b2.reshape(1, D), W3)
    acc = _edge_call(h2, src, dst).reshape(NC, NPAD, D)

    emb, logits, probs = _head(acc, h2, dinv_n, b3.reshape(1, D),
                               batch.reshape(_NRB, _RB),
                               Wm0, bm0.reshape(1, H), Wm1, bm1.reshape(1, OUT))
    return (logits, probs, emb)


# speedup vs baseline: 1.0744x; 1.0744x over previous
import functools

import jax
import jax.numpy as jnp
from jax import lax
from jax.experimental import pallas as pl
from jax.experimental.pallas import tpu as pltpu
from jax.experimental.pallas import tpu_sc as plsc

N = 10000
NPAD = 10240
E = 320000
D = 128
G = 64
H = 64
OUT = 10

NC = 2
NS = 16
NW = NC * NS
KB = 80
NB = 125
EPW = NB * KB
RPT = NPAD // NS

_mesh = plsc.VectorSubcoreMesh(core_axis_name="c", subcore_axis_name="s")


def _sc_deg_body(edge_hbm, out_hbm, acc_sh, dst_v, ones_v, zero_v, sem_i):
    c = lax.axis_index("c")
    s = lax.axis_index("s")
    wid = c * NS + s
    fetch = pltpu.async_copy(edge_hbm.at[1, wid], dst_v, sem_i)
    for i in range(KB // 16):
        ones_v[pl.ds(i * 16, 16)] = jnp.ones((16,), jnp.float32)
    for i in range(8):
        zero_v[pl.ds(i * 16, 16)] = jnp.zeros((16,), jnp.float32)

    def zloop(j, carry):
        pltpu.sync_copy(zero_v, acc_sh.at[pl.ds(s * RPT + j * 128, 128)])
        return carry

    lax.fori_loop(0, RPT // 128, zloop, 0)
    fetch.wait()
    plsc.subcore_barrier()
    adds = [pltpu.async_copy(ones_v, acc_sh.at[dst_v.at[j]], sem_i, add=True)
            for j in range(NB)]
    for a in adds:
        a.wait()
    plsc.subcore_barrier()
    pltpu.sync_copy(acc_sh.at[pl.ds(s * RPT, RPT)],
                    out_hbm.at[pl.ds(c * NPAD + s * RPT, RPT)])


_deg_call = pl.kernel(
    _sc_deg_body,
    mesh=_mesh,
    out_type=jax.ShapeDtypeStruct((NC * NPAD,), jnp.float32),
    scratch_types=[
        pltpu.VMEM_SHARED((NPAD,), jnp.float32),
        pltpu.VMEM((NB, KB), jnp.int32),
        pltpu.VMEM((KB,), jnp.float32),
        pltpu.VMEM((128,), jnp.float32),
        pltpu.SemaphoreType.DMA,
    ],
)


_ZR = 8
_RB4 = 4
_RI = 8


def _sc_edge_body(h2_hbm, edge_hbm, out_hbm, acc_sh,
                  src_v, dst_v, r0, r1, r2, r3, zrows_v,
                  g0, g1, g2, g3,
                  i0, i1, i2, i3, i4, i5, i6, i7, sem_z):
    c = lax.axis_index("c")
    s = lax.axis_index("s")
    wid = c * NS + s
    base = wid * EPW
    rows = (r0, r1, r2, r3)
    gsems = (g0, g1, g2, g3)
    isems = (i0, i1, i2, i3, i4, i5, i6, i7)

    def idx_issue(j, q):
        pltpu.async_copy(edge_hbm.at[0, wid, pl.ds(j, 1)],
                         src_v.at[pl.ds(q, 1)], isems[q])
        pltpu.async_copy(edge_hbm.at[1, wid, pl.ds(j, 1)],
                         dst_v.at[pl.ds(q, 1)], isems[q])

    def idx_wait(q):
        pltpu.make_async_copy(edge_hbm.at[0, wid, pl.ds(0, 1)],
                              src_v.at[pl.ds(q, 1)], isems[q]).wait()
        pltpu.make_async_copy(edge_hbm.at[1, wid, pl.ds(0, 1)],
                              dst_v.at[pl.ds(q, 1)], isems[q]).wait()

    for q in range(_RI):
        idx_issue(q, q)

    for r in range(_ZR):
        for q in range(D // 16):
            zrows_v[r, pl.ds(q * 16, 16)] = jnp.zeros((16,), jnp.float32)
    zcopies = [
        pltpu.async_copy(zrows_v, acc_sh.at[pl.ds(s * RPT + j * _ZR, _ZR)],
                         sem_z)
        for j in range(RPT // _ZR)
    ]
    for zc in zcopies:
        zc.wait()
    plsc.subcore_barrier()

    for j in range(3):
        idx_wait(j)
        pltpu.async_copy(h2_hbm.at[src_v.at[j]], rows[j], gsems[j])

    def eloop(i, carry):
        for qq in range(_RI):
            j = _RI * i + qq

            @pl.when(j < NB)
            def _body():
                p = qq % _RB4
                p3 = (qq + 3) % _RB4
                q3 = (qq + 3) % _RI

                pltpu.make_async_copy(h2_hbm.at[src_v.at[qq]],
                                      rows[p], gsems[p]).wait()
                pltpu.sync_copy(rows[p], acc_sh.at[dst_v.at[qq]], add=True)

                @pl.when(j + _RI < NB)
                def _reidx():
                    idx_issue(j + _RI, qq)

                @pl.when(j + 3 < NB)
                def _regather():
                    idx_wait(q3)
                    pltpu.async_copy(h2_hbm.at[src_v.at[q3]], rows[p3],
                                     gsems[p3])
        return carry

    lax.fori_loop(0, (NB + _RI - 1) // _RI, eloop, 0)
    plsc.subcore_barrier()
    pltpu.sync_copy(acc_sh.at[pl.ds(s * RPT, RPT)],
                    out_hbm.at[pl.ds(c * NPAD + s * RPT, RPT)])


_edge_call = pl.kernel(
    _sc_edge_body,
    mesh=_mesh,
    out_type=jax.ShapeDtypeStruct((NC * NPAD, D), jnp.float32),
    scratch_types=[
        pltpu.VMEM_SHARED((NPAD, D), jnp.float32),
        pltpu.VMEM((_RI, KB), jnp.int32),
        pltpu.VMEM((_RI, KB), jnp.int32),
        pltpu.VMEM((KB, D), jnp.float32),
        pltpu.VMEM((KB, D), jnp.float32),
        pltpu.VMEM((KB, D), jnp.float32),
        pltpu.VMEM((KB, D), jnp.float32),
        pltpu.VMEM((_ZR, D), jnp.float32),
    ] + [pltpu.SemaphoreType.DMA] * 13,
)


def _dinv_body(indeg_ref, out_ref):
    blk = indeg_ref[...]
    dsum = 1.0 + blk[0] + blk[1]
    dinv = lax.rsqrt(dsum)
    ones = jnp.ones((1, 128), jnp.float32)
    for r in range(8):
        out_ref[pl.ds(r * 128, 128), :] = lax.dot_general(
            dinv[r:r + 1, :], ones, (((0,), (0,)), ((), ())),
            preferred_element_type=jnp.float32)


def _dinv_call(indeg):
    return pl.pallas_call(
        _dinv_body,
        grid=(NPAD // 1024,),
        in_specs=[pl.BlockSpec((2, 8, 128), lambda i: (0, i, 0))],
        out_specs=pl.BlockSpec((1024, 128), lambda i: (i, 0)),
        out_shape=jax.ShapeDtypeStruct((NPAD, D), jnp.float32),
    )(indeg)


_RB = 1000
_NRB = N // _RB


def _mm_scale_body(x_ref, w_ref, dinv_ref, o_ref):
    o_ref[...] = jnp.dot(x_ref[...], w_ref[...],
                         preferred_element_type=jnp.float32) * dinv_ref[...]


def _mm_scale(x, w, dinv_b):
    return pl.pallas_call(
        _mm_scale_body,
        grid=(_NRB,),
        in_specs=[
            pl.BlockSpec((_RB, D), lambda i: (i, 0)),
            pl.BlockSpec((D, D), lambda i: (0, 0)),
            pl.BlockSpec((_RB, D), lambda i: (i, 0)),
        ],
        out_specs=pl.BlockSpec((_RB, D), lambda i: (i, 0)),
        out_shape=jax.ShapeDtypeStruct((N, D), jnp.float32),
    )(x, w, dinv_b)


def _layer_body(acc_ref, h2_ref, dinv_ref, b_ref, w_ref, o_ref):
    a = acc_ref[...]
    dinv = dinv_ref[...]
    pre = dinv * (a[0] + a[1] + h2_ref[...]) + b_ref[...]
    xn = jnp.maximum(pre, 0.0)
    o_ref[...] = jnp.dot(xn, w_ref[...],
                         preferred_element_type=jnp.float32) * dinv


def _layer(acc, h2, dinv_b, b, w):
    return pl.pallas_call(
        _layer_body,
        grid=(_NRB,),
        in_specs=[
            pl.BlockSpec((2, _RB, D), lambda i: (0, i, 0)),
            pl.BlockSpec((_RB, D), lambda i: (i, 0)),
            pl.BlockSpec((_RB, D), lambda i: (i, 0)),
            pl.BlockSpec((1, D), lambda i: (0, 0)),
            pl.BlockSpec((D, D), lambda i: (0, 0)),
        ],
        out_specs=pl.BlockSpec((_RB, D), lambda i: (i, 0)),
        out_shape=jax.ShapeDtypeStruct((N, D), jnp.float32),
    )(acc, h2, dinv_b, b, w)


def _head_body(acc_ref, h2_ref, dinv_ref, b_ref, batch_ref,
               wm0_ref, bm0_ref, wm1_ref, bm1_ref,
               emb_ref, logits_ref, probs_ref,
               pooled_ref, counts_ref):
    i = pl.program_id(0)
    a = acc_ref[...]
    pre = dinv_ref[...] * (a[0] + a[1] + h2_ref[...]) + b_ref[...]
    emb = jnp.maximum(pre, 0.0)
    emb_ref[...] = emb

    bat = batch_ref[pl.ds(i, 1), :]
    gids = lax.broadcasted_iota(jnp.int32, (G, 1), 0)
    onehot = (bat == gids).astype(jnp.float32)

    @pl.when(i == 0)
    def _init():
        pooled_ref[...] = jnp.zeros((G, D), jnp.float32)
        counts_ref[...] = jnp.zeros((G, D), jnp.float32)

    pooled_ref[...] += jnp.dot(onehot, emb, preferred_element_type=jnp.float32)
    cnt = jnp.sum(onehot, axis=1, keepdims=True)
    counts_ref[...] += jnp.broadcast_to(cnt, (G, D))

    @pl.when(i == _NRB - 1)
    def _final():
        pooled = pooled_ref[...] / jnp.maximum(counts_ref[...], 1.0)
        z = jnp.dot(pooled, wm0_ref[...],
                    preferred_element_type=jnp.float32) + bm0_ref[...]
        z = jnp.where(z > 0.0, z, jnp.exp(jnp.minimum(z, 0.0)) - 1.0)
        logits = jnp.dot(z, wm1_ref[...],
                         preferred_element_type=jnp.float32) + bm1_ref[...]
        logits_ref[...] = logits
        m = jnp.max(logits, axis=-1, keepdims=True)
        e = jnp.exp(logits - m)
        probs_ref[...] = e / jnp.sum(e, axis=-1, keepdims=True)


def _head(acc, h2, dinv_b, b, batch2d, wm0, bm0, wm1, bm1):
    return pl.pallas_call(
        _head_body,
        grid=(_NRB,),
        in_specs=[
            pl.BlockSpec((2, _RB, D), lambda i: (0, i, 0)),
            pl.BlockSpec((_RB, D), lambda i: (i, 0)),
            pl.BlockSpec((_RB, D), lambda i: (i, 0)),
            pl.BlockSpec((1, D), lambda i: (0, 0)),
            pl.BlockSpec((_NRB, _RB), lambda i: (0, 0)),
            pl.BlockSpec((D, H), lambda i: (0, 0)),
            pl.BlockSpec((1, H), lambda i: (0, 0)),
            pl.BlockSpec((H, OUT), lambda i: (0, 0)),
            pl.BlockSpec((1, OUT), lambda i: (0, 0)),
        ],
        out_specs=[
            pl.BlockSpec((_RB, D), lambda i: (i, 0)),
            pl.BlockSpec((G, OUT), lambda i: (0, 0)),
            pl.BlockSpec((G, OUT), lambda i: (0, 0)),
        ],
        out_shape=[
            jax.ShapeDtypeStruct((N, D), jnp.float32),
            jax.ShapeDtypeStruct((G, OUT), jnp.float32),
            jax.ShapeDtypeStruct((G, OUT), jnp.float32),
        ],
        scratch_shapes=[
            pltpu.VMEM((G, D), jnp.float32),
            pltpu.VMEM((G, D), jnp.float32),
        ],
    )(acc, h2, dinv_b, b, batch2d, wm0, bm0, wm1, bm1)


def kernel(x, edge_index, batch, W1, b1, W2, b2, W3, b3, Wm0, bm0, Wm1, bm1):
    edge3 = edge_index.reshape(2, NW, NB, KB)

    indeg = _deg_call(edge3).reshape(NC, NPAD // 128, 128)
    dinv_n = _dinv_call(indeg)

    h2 = _mm_scale(x, W1, dinv_n)
    acc = _edge_call(h2, edge3).reshape(NC, NPAD, D)
    h2 = _layer(acc, h2, dinv_n, b1.reshape(1, D), W2)
    acc = _edge_call(h2, edge3).reshape(NC, NPAD, D)
    h2 = _layer(acc, h2, dinv_n, b2.reshape(1, D), W3)
    acc = _edge_call(h2, edge3).reshape(NC, NPAD, D)

    emb, logits, probs = _head(acc, h2, dinv_n, b3.reshape(1, D),
                               batch.reshape(_NRB, _RB),
                               Wm0, bm0.reshape(1, H), Wm1, bm1.reshape(1, OUT))
    return (logits, probs, emb)


# speedup vs baseline: 1.0850x; 1.0098x over previous
import functools

import jax
import jax.numpy as jnp
from jax import lax
from jax.experimental import pallas as pl
from jax.experimental.pallas import tpu as pltpu
from jax.experimental.pallas import tpu_sc as plsc

N = 10000
NPAD = 10240
E = 320000
D = 128
G = 64
H = 64
OUT = 10

NC = 2
NS = 16
NW = NC * NS
KB = 40
NB = 250
EPW = NB * KB
RPT = NPAD // NS

_mesh = plsc.VectorSubcoreMesh(core_axis_name="c", subcore_axis_name="s")


def _sc_deg_body(edge_hbm, out_hbm, acc_sh, dst_v, ones_v, zero_v, sem_i):
    c = lax.axis_index("c")
    s = lax.axis_index("s")
    wid = c * NS + s
    fetch = pltpu.async_copy(edge_hbm.at[1, wid], dst_v, sem_i)
    for i in range(3):
        ones_v[pl.ds(i * 16, 16)] = jnp.ones((16,), jnp.float32)
    for i in range(8):
        zero_v[pl.ds(i * 16, 16)] = jnp.zeros((16,), jnp.float32)

    def zloop(j, carry):
        pltpu.sync_copy(zero_v, acc_sh.at[pl.ds(s * RPT + j * 128, 128)])
        return carry

    lax.fori_loop(0, RPT // 128, zloop, 0)
    fetch.wait()
    plsc.subcore_barrier()
    adds = [pltpu.async_copy(ones_v.at[pl.ds(0, KB)], acc_sh.at[dst_v.at[j]],
                             sem_i, add=True)
            for j in range(NB)]
    for a in adds:
        a.wait()
    plsc.subcore_barrier()
    pltpu.sync_copy(acc_sh.at[pl.ds(s * RPT, RPT)],
                    out_hbm.at[pl.ds(c * NPAD + s * RPT, RPT)])


_deg_call = pl.kernel(
    _sc_deg_body,
    mesh=_mesh,
    out_type=jax.ShapeDtypeStruct((NC * NPAD,), jnp.float32),
    scratch_types=[
        pltpu.VMEM_SHARED((NPAD,), jnp.float32),
        pltpu.VMEM((NB, KB), jnp.int32),
        pltpu.VMEM((48,), jnp.float32),
        pltpu.VMEM((128,), jnp.float32),
        pltpu.SemaphoreType.DMA,
    ],
)


_ZR = 8
_NRW = 6
_RI = 12


def _sc_edge_body(h2_hbm, edge_hbm, out_hbm, acc_sh,
                  src_v, dst_v, r0, r1, r2, r3, r4, r5, zrows_v,
                  g0, g1, g2, g3, g4, g5,
                  i0, i1, i2, i3, i4, i5, i6, i7, i8, i9, ia, ib, sem_z):
    c = lax.axis_index("c")
    s = lax.axis_index("s")
    wid = c * NS + s
    rows = (r0, r1, r2, r3, r4, r5)
    gsems = (g0, g1, g2, g3, g4, g5)
    isems = (i0, i1, i2, i3, i4, i5, i6, i7, i8, i9, ia, ib)

    def idx_issue(j, q):
        pltpu.async_copy(edge_hbm.at[0, wid, pl.ds(j, 1)],
                         src_v.at[pl.ds(q, 1)], isems[q])
        pltpu.async_copy(edge_hbm.at[1, wid, pl.ds(j, 1)],
                         dst_v.at[pl.ds(q, 1)], isems[q])

    def idx_wait(q):
        pltpu.make_async_copy(edge_hbm.at[0, wid, pl.ds(0, 1)],
                              src_v.at[pl.ds(q, 1)], isems[q]).wait()
        pltpu.make_async_copy(edge_hbm.at[1, wid, pl.ds(0, 1)],
                              dst_v.at[pl.ds(q, 1)], isems[q]).wait()

    for q in range(_RI):
        idx_issue(q, q)

    for r in range(_ZR):
        for q in range(D // 16):
            zrows_v[r, pl.ds(q * 16, 16)] = jnp.zeros((16,), jnp.float32)
    zcopies = [
        pltpu.async_copy(zrows_v, acc_sh.at[pl.ds(s * RPT + j * _ZR, _ZR)],
                         sem_z)
        for j in range(RPT // _ZR)
    ]
    for zc in zcopies:
        zc.wait()
    plsc.subcore_barrier()

    for j in range(_NRW - 1):
        idx_wait(j)
        pltpu.async_copy(h2_hbm.at[src_v.at[j]], rows[j], gsems[j])

    def eloop(i, carry):
        for qq in range(_RI):
            j = _RI * i + qq

            @pl.when(j < NB)
            def _body():
                p = qq % _NRW
                p5 = (qq + _NRW - 1) % _NRW
                q5 = (qq + _NRW - 1) % _RI

                pltpu.make_async_copy(h2_hbm.at[src_v.at[qq]],
                                      rows[p], gsems[p]).wait()
                pltpu.sync_copy(rows[p], acc_sh.at[dst_v.at[qq]], add=True)

                @pl.when(j + _RI < NB)
                def _reidx():
                    idx_issue(j + _RI, qq)

                @pl.when(j + _NRW - 1 < NB)
                def _regather():
                    idx_wait(q5)
                    pltpu.async_copy(h2_hbm.at[src_v.at[q5]], rows[p5],
                                     gsems[p5])
        return carry

    lax.fori_loop(0, (NB + _RI - 1) // _RI, eloop, 0)
    plsc.subcore_barrier()
    pltpu.sync_copy(acc_sh.at[pl.ds(s * RPT, RPT)],
                    out_hbm.at[pl.ds(c * NPAD + s * RPT, RPT)])


_edge_call = pl.kernel(
    _sc_edge_body,
    mesh=_mesh,
    out_type=jax.ShapeDtypeStruct((NC * NPAD, D), jnp.float32),
    scratch_types=[
        pltpu.VMEM_SHARED((NPAD, D), jnp.float32),
        pltpu.VMEM((_RI, KB), jnp.int32),
        pltpu.VMEM((_RI, KB), jnp.int32),
        pltpu.VMEM((KB, D), jnp.float32),
        pltpu.VMEM((KB, D), jnp.float32),
        pltpu.VMEM((KB, D), jnp.float32),
        pltpu.VMEM((KB, D), jnp.float32),
        pltpu.VMEM((KB, D), jnp.float32),
        pltpu.VMEM((KB, D), jnp.float32),
        pltpu.VMEM((_ZR, D), jnp.float32),
    ] + [pltpu.SemaphoreType.DMA] * 19,
)


def _dinv_body(indeg_ref, out_ref):
    blk = indeg_ref[...]
    dsum = 1.0 + blk[0] + blk[1]
    dinv = lax.rsqrt(dsum)
    ones = jnp.ones((1, 128), jnp.float32)
    for r in range(8):
        out_ref[pl.ds(r * 128, 128), :] = lax.dot_general(
            dinv[r:r + 1, :], ones, (((0,), (0,)), ((), ())),
            preferred_element_type=jnp.float32)


def _dinv_call(indeg):
    return pl.pallas_call(
        _dinv_body,
        grid=(NPAD // 1024,),
        in_specs=[pl.BlockSpec((2, 8, 128), lambda i: (0, i, 0))],
        out_specs=pl.BlockSpec((1024, 128), lambda i: (i, 0)),
        out_shape=jax.ShapeDtypeStruct((NPAD, D), jnp.float32),
    )(indeg)


_RB = 1000
_NRB = N // _RB


def _mm_scale_body(x_ref, w_ref, dinv_ref, o_ref):
    o_ref[...] = jnp.dot(x_ref[...], w_ref[...],
                         preferred_element_type=jnp.float32) * dinv_ref[...]


def _mm_scale(x, w, dinv_b):
    return pl.pallas_call(
        _mm_scale_body,
        grid=(_NRB,),
        in_specs=[
            pl.BlockSpec((_RB, D), lambda i: (i, 0)),
            pl.BlockSpec((D, D), lambda i: (0, 0)),
            pl.BlockSpec((_RB, D), lambda i: (i, 0)),
        ],
        out_specs=pl.BlockSpec((_RB, D), lambda i: (i, 0)),
        out_shape=jax.ShapeDtypeStruct((N, D), jnp.float32),
    )(x, w, dinv_b)


def _layer_body(acc_ref, h2_ref, dinv_ref, b_ref, w_ref, o_ref):
    a = acc_ref[...]
    dinv = dinv_ref[...]
    pre = dinv * (a[0] + a[1] + h2_ref[...]) + b_ref[...]
    xn = jnp.maximum(pre, 0.0)
    o_ref[...] = jnp.dot(xn, w_ref[...],
                         preferred_element_type=jnp.float32) * dinv


def _layer(acc, h2, dinv_b, b, w):
    return pl.pallas_call(
        _layer_body,
        grid=(_NRB,),
        in_specs=[
            pl.BlockSpec((2, _RB, D), lambda i: (0, i, 0)),
            pl.BlockSpec((_RB, D), lambda i: (i, 0)),
            pl.BlockSpec((_RB, D), lambda i: (i, 0)),
            pl.BlockSpec((1, D), lambda i: (0, 0)),
            pl.BlockSpec((D, D), lambda i: (0, 0)),
        ],
        out_specs=pl.BlockSpec((_RB, D), lambda i: (i, 0)),
        out_shape=jax.ShapeDtypeStruct((N, D), jnp.float32),
    )(acc, h2, dinv_b, b, w)


def _head_body(acc_ref, h2_ref, dinv_ref, b_ref, batch_ref,
               wm0_ref, bm0_ref, wm1_ref, bm1_ref,
               emb_ref, logits_ref, probs_ref,
               pooled_ref, counts_ref):
    i = pl.program_id(0)
    a = acc_ref[...]
    pre = dinv_ref[...] * (a[0] + a[1] + h2_ref[...]) + b_ref[...]
    emb = jnp.maximum(pre, 0.0)
    emb_ref[...] = emb

    bat = batch_ref[pl.ds(i, 1), :]
    gids = lax.broadcasted_iota(jnp.int32, (G, 1), 0)
    onehot = (bat == gids).astype(jnp.float32)

    @pl.when(i == 0)
    def _init():
        pooled_ref[...] = jnp.zeros((G, D), jnp.float32)
        counts_ref[...] = jnp.zeros((G, D), jnp.float32)

    pooled_ref[...] += jnp.dot(onehot, emb, preferred_element_type=jnp.float32)
    cnt = jnp.sum(onehot, axis=1, keepdims=True)
    counts_ref[...] += jnp.broadcast_to(cnt, (G, D))

    @pl.when(i == _NRB - 1)
    def _final():
        pooled = pooled_ref[...] / jnp.maximum(counts_ref[...], 1.0)
        z = jnp.dot(pooled, wm0_ref[...],
                    preferred_element_type=jnp.float32) + bm0_ref[...]
        z = jnp.where(z > 0.0, z, jnp.exp(jnp.minimum(z, 0.0)) - 1.0)
        logits = jnp.dot(z, wm1_ref[...],
                         preferred_element_type=jnp.float32) + bm1_ref[...]
        logits_ref[...] = logits
        m = jnp.max(logits, axis=-1, keepdims=True)
        e = jnp.exp(logits - m)
        probs_ref[...] = e / jnp.sum(e, axis=-1, keepdims=True)


def _head(acc, h2, dinv_b, b, batch2d, wm0, bm0, wm1, bm1):
    return pl.pallas_call(
        _head_body,
        grid=(_NRB,),
        in_specs=[
            pl.BlockSpec((2, _RB, D), lambda i: (0, i, 0)),
            pl.BlockSpec((_RB, D), lambda i: (i, 0)),
            pl.BlockSpec((_RB, D), lambda i: (i, 0)),
            pl.BlockSpec((1, D), lambda i: (0, 0)),
            pl.BlockSpec((_NRB, _RB), lambda i: (0, 0)),
            pl.BlockSpec((D, H), lambda i: (0, 0)),
            pl.BlockSpec((1, H), lambda i: (0, 0)),
            pl.BlockSpec((H, OUT), lambda i: (0, 0)),
            pl.BlockSpec((1, OUT), lambda i: (0, 0)),
        ],
        out_specs=[
            pl.BlockSpec((_RB, D), lambda i: (i, 0)),
            pl.BlockSpec((G, OUT), lambda i: (0, 0)),
            pl.BlockSpec((G, OUT), lambda i: (0, 0)),
        ],
        out_shape=[
            jax.ShapeDtypeStruct((N, D), jnp.float32),
            jax.ShapeDtypeStruct((G, OUT), jnp.float32),
            jax.ShapeDtypeStruct((G, OUT), jnp.float32),
        ],
        scratch_shapes=[
            pltpu.VMEM((G, D), jnp.float32),
            pltpu.VMEM((G, D), jnp.float32),
        ],
    )(acc, h2, dinv_b, b, batch2d, wm0, bm0, wm1, bm1)


def kernel(x, edge_index, batch, W1, b1, W2, b2, W3, b3, Wm0, bm0, Wm1, bm1):
    edge3 = edge_index.reshape(2, NW, NB, KB)

    indeg = _deg_call(edge3).reshape(NC, NPAD // 128, 128)
    dinv_n = _dinv_call(indeg)

    h2 = _mm_scale(x, W1, dinv_n)
    acc = _edge_call(h2, edge3).reshape(NC, NPAD, D)
    h2 = _layer(acc, h2, dinv_n, b1.reshape(1, D), W2)
    acc = _edge_call(h2, edge3).reshape(NC, NPAD, D)
    h2 = _layer(acc, h2, dinv_n, b2.reshape(1, D), W3)
    acc = _edge_call(h2, edge3).reshape(NC, NPAD, D)

    emb, logits, probs = _head(acc, h2, dinv_n, b3.reshape(1, D),
                               batch.reshape(_NRB, _RB),
                               Wm0, bm0.reshape(1, H), Wm1, bm1.reshape(1, OUT))
    return (logits, probs, emb)


# speedup vs baseline: 1.0859x; 1.0009x over previous
import jax
import jax.numpy as jnp
from jax import lax
from jax.experimental import pallas as pl
from jax.experimental.pallas import tpu as pltpu
from jax.experimental.pallas import tpu_sc as plsc

N = 10000
NPAD = 10240
E = 320000
D = 128
G = 64
H = 64
OUT = 10

NC = 2
NS = 16
NW = NC * NS
KB = 40
NB = 250
RPT = NPAD // NS

_mesh = plsc.VectorSubcoreMesh(core_axis_name="c", subcore_axis_name="s")


def _sc_deg_body(edge_hbm, out_hbm, acc_sh, dst_v, ones_v, zero_v, sem_i):
    c = lax.axis_index("c")
    s = lax.axis_index("s")
    wid = c * NS + s
    fetch = pltpu.async_copy(edge_hbm.at[1, wid], dst_v, sem_i)
    for i in range(3):
        ones_v[pl.ds(i * 16, 16)] = jnp.ones((16,), jnp.float32)
    for i in range(8):
        zero_v[pl.ds(i * 16, 16)] = jnp.zeros((16,), jnp.float32)

    def zloop(j, carry):
        pltpu.sync_copy(zero_v, acc_sh.at[pl.ds(s * RPT + j * 128, 128)])
        return carry

    lax.fori_loop(0, RPT // 128, zloop, 0)
    fetch.wait()
    plsc.subcore_barrier()
    adds = [pltpu.async_copy(ones_v.at[pl.ds(0, KB)], acc_sh.at[dst_v.at[j]],
                             sem_i, add=True)
            for j in range(NB)]
    for a in adds:
        a.wait()
    plsc.subcore_barrier()
    pltpu.sync_copy(acc_sh.at[pl.ds(s * RPT, RPT)],
                    out_hbm.at[pl.ds(c * NPAD + s * RPT, RPT)])


_deg_call = pl.kernel(
    _sc_deg_body,
    mesh=_mesh,
    out_type=jax.ShapeDtypeStruct((NC * NPAD,), jnp.float32),
    scratch_types=[
        pltpu.VMEM_SHARED((NPAD,), jnp.float32),
        pltpu.VMEM((NB, KB), jnp.int32),
        pltpu.VMEM((48,), jnp.float32),
        pltpu.VMEM((128,), jnp.float32),
        pltpu.SemaphoreType.DMA,
    ],
)


_ZR = 8
_NRW = 6
_RI = 12


def _sc_edge_body(h2_hbm, edge_hbm, out_hbm, acc_sh,
                  src_v, dst_v, r0, r1, r2, r3, r4, r5, zrows_v,
                  g0, g1, g2, g3, g4, g5,
                  i0, i1, i2, i3, i4, i5, i6, i7, i8, i9, ia, ib, sem_z):
    c = lax.axis_index("c")
    s = lax.axis_index("s")
    wid = c * NS + s
    rows = (r0, r1, r2, r3, r4, r5)
    gsems = (g0, g1, g2, g3, g4, g5)
    isems = (i0, i1, i2, i3, i4, i5, i6, i7, i8, i9, ia, ib)

    def idx_issue(j, q):
        pltpu.async_copy(edge_hbm.at[0, wid, pl.ds(j, 1)],
                         src_v.at[pl.ds(q, 1)], isems[q])
        pltpu.async_copy(edge_hbm.at[1, wid, pl.ds(j, 1)],
                         dst_v.at[pl.ds(q, 1)], isems[q])

    def idx_wait(q):
        pltpu.make_async_copy(edge_hbm.at[0, wid, pl.ds(0, 1)],
                              src_v.at[pl.ds(q, 1)], isems[q]).wait()
        pltpu.make_async_copy(edge_hbm.at[1, wid, pl.ds(0, 1)],
                              dst_v.at[pl.ds(q, 1)], isems[q]).wait()

    for q in range(_RI):
        idx_issue(q, q)

    for r in range(_ZR):
        for q in range(D // 16):
            zrows_v[r, pl.ds(q * 16, 16)] = jnp.zeros((16,), jnp.float32)
    zcopies = [
        pltpu.async_copy(zrows_v, acc_sh.at[pl.ds(s * RPT + j * _ZR, _ZR)],
                         sem_z)
        for j in range(RPT // _ZR)
    ]
    for zc in zcopies:
        zc.wait()
    plsc.subcore_barrier()

    for j in range(_NRW - 1):
        idx_wait(j)
        pltpu.async_copy(h2_hbm.at[src_v.at[j]], rows[j], gsems[j])

    def eloop(i, carry):
        for qq in range(_RI):
            j = _RI * i + qq

            @pl.when(j < NB)
            def _body():
                p = qq % _NRW
                p5 = (qq + _NRW - 1) % _NRW
                q5 = (qq + _NRW - 1) % _RI

                pltpu.make_async_copy(h2_hbm.at[src_v.at[qq]],
                                      rows[p], gsems[p]).wait()
                pltpu.sync_copy(rows[p], acc_sh.at[dst_v.at[qq]], add=True)

                @pl.when(j + _RI < NB)
                def _reidx():
                    idx_issue(j + _RI, qq)

                @pl.when(j + _NRW - 1 < NB)
                def _regather():
                    idx_wait(q5)
                    pltpu.async_copy(h2_hbm.at[src_v.at[q5]], rows[p5],
                                     gsems[p5])
        return carry

    lax.fori_loop(0, (NB + _RI - 1) // _RI, eloop, 0)
    plsc.subcore_barrier()
    pltpu.sync_copy(acc_sh.at[pl.ds(s * RPT, RPT)],
                    out_hbm.at[pl.ds(c * NPAD + s * RPT, RPT)])


_edge_call = pl.kernel(
    _sc_edge_body,
    mesh=_mesh,
    out_type=jax.ShapeDtypeStruct((NC * NPAD, D), jnp.float32),
    scratch_types=[
        pltpu.VMEM_SHARED((NPAD, D), jnp.float32),
        pltpu.VMEM((_RI, KB), jnp.int32),
        pltpu.VMEM((_RI, KB), jnp.int32),
        pltpu.VMEM((KB, D), jnp.float32),
        pltpu.VMEM((KB, D), jnp.float32),
        pltpu.VMEM((KB, D), jnp.float32),
        pltpu.VMEM((KB, D), jnp.float32),
        pltpu.VMEM((KB, D), jnp.float32),
        pltpu.VMEM((KB, D), jnp.float32),
        pltpu.VMEM((_ZR, D), jnp.float32),
    ] + [pltpu.SemaphoreType.DMA] * 19,
)


def _dinv_body(indeg_ref, out_ref):
    blk = indeg_ref[...]
    dsum = 1.0 + blk[0] + blk[1]
    dinv = lax.rsqrt(dsum)
    ones = jnp.ones((1, 128), jnp.float32)
    for r in range(8):
        out_ref[pl.ds(r * 128, 128), :] = lax.dot_general(
            dinv[r:r + 1, :], ones, (((0,), (0,)), ((), ())),
            preferred_element_type=jnp.float32)


def _dinv_call(indeg):
    return pl.pallas_call(
        _dinv_body,
        grid=(NPAD // 1024,),
        in_specs=[pl.BlockSpec((2, 8, 128), lambda i: (0, i, 0))],
        out_specs=pl.BlockSpec((1024, 128), lambda i: (i, 0)),
        out_shape=jax.ShapeDtypeStruct((NPAD, D), jnp.float32),
    )(indeg)


_RB = 1000
_NRB = N // _RB


def _mm_scale_body(x_ref, w_ref, dinv_ref, o_ref):
    o_ref[...] = jnp.dot(x_ref[...], w_ref[...],
                         preferred_element_type=jnp.float32) * dinv_ref[...]


def _mm_scale(x, w, dinv_b):
    return pl.pallas_call(
        _mm_scale_body,
        grid=(_NRB,),
        in_specs=[
            pl.BlockSpec((_RB, D), lambda i: (i, 0)),
            pl.BlockSpec((D, D), lambda i: (0, 0)),
            pl.BlockSpec((_RB, D), lambda i: (i, 0)),
        ],
        out_specs=pl.BlockSpec((_RB, D), lambda i: (i, 0)),
        out_shape=jax.ShapeDtypeStruct((N, D), jnp.float32),
    )(x, w, dinv_b)


def _layer_body(acc_ref, h2_ref, dinv_ref, b_ref, w_ref, o_ref):
    a = acc_ref[...]
    dinv = dinv_ref[...]
    pre = dinv * (a[0] + a[1] + h2_ref[...]) + b_ref[...]
    xn = jnp.maximum(pre, 0.0)
    o_ref[...] = jnp.dot(xn, w_ref[...],
                         preferred_element_type=jnp.float32) * dinv


def _layer(acc, h2, dinv_b, b, w):
    return pl.pallas_call(
        _layer_body,
        grid=(_NRB,),
        in_specs=[
            pl.BlockSpec((2, _RB, D), lambda i: (0, i, 0)),
            pl.BlockSpec((_RB, D), lambda i: (i, 0)),
            pl.BlockSpec((_RB, D), lambda i: (i, 0)),
            pl.BlockSpec((1, D), lambda i: (0, 0)),
            pl.BlockSpec((D, D), lambda i: (0, 0)),
        ],
        out_specs=pl.BlockSpec((_RB, D), lambda i: (i, 0)),
        out_shape=jax.ShapeDtypeStruct((N, D), jnp.float32),
    )(acc, h2, dinv_b, b, w)


def _head_body(acc_ref, h2_ref, dinv_ref, b_ref, batch_ref,
               wm0_ref, bm0_ref, wm1_ref, bm1_ref,
               emb_ref, logits_ref, probs_ref,
               pooled_ref, counts_ref):
    i = pl.program_id(0)
    a = acc_ref[...]
    pre = dinv_ref[...] * (a[0] + a[1] + h2_ref[...]) + b_ref[...]
    emb = jnp.maximum(pre, 0.0)
    emb_ref[...] = emb

    bat = batch_ref[pl.ds(i, 1), :]
    gids = lax.broadcasted_iota(jnp.int32, (G, 1), 0)
    onehot = (bat == gids).astype(jnp.float32)

    @pl.when(i == 0)
    def _init():
        pooled_ref[...] = jnp.zeros((G, D), jnp.float32)
        counts_ref[...] = jnp.zeros((G, D), jnp.float32)

    pooled_ref[...] += jnp.dot(onehot, emb, preferred_element_type=jnp.float32)
    cnt = jnp.sum(onehot, axis=1, keepdims=True)
    counts_ref[...] += jnp.broadcast_to(cnt, (G, D))

    @pl.when(i == _NRB - 1)
    def _final():
        pooled = pooled_ref[...] / jnp.maximum(counts_ref[...], 1.0)
        z = jnp.dot(pooled, wm0_ref[...],
                    preferred_element_type=jnp.float32) + bm0_ref[...]
        z = jnp.where(z > 0.0, z, jnp.exp(jnp.minimum(z, 0.0)) - 1.0)
        logits = jnp.dot(z, wm1_ref[...],
                         preferred_element_type=jnp.float32) + bm1_ref[...]
        logits_ref[...] = logits
        m = jnp.max(logits, axis=-1, keepdims=True)
        e = jnp.exp(logits - m)
        probs_ref[...] = e / jnp.sum(e, axis=-1, keepdims=True)


def _head(acc, h2, dinv_b, b, batch2d, wm0, bm0, wm1, bm1):
    return pl.pallas_call(
        _head_body,
        grid=(_NRB,),
        in_specs=[
            pl.BlockSpec((2, _RB, D), lambda i: (0, i, 0)),
            pl.BlockSpec((_RB, D), lambda i: (i, 0)),
            pl.BlockSpec((_RB, D), lambda i: (i, 0)),
            pl.BlockSpec((1, D), lambda i: (0, 0)),
            pl.BlockSpec((_NRB, _RB), lambda i: (0, 0)),
            pl.BlockSpec((D, H), lambda i: (0, 0)),
            pl.BlockSpec((1, H), lambda i: (0, 0)),
            pl.BlockSpec((H, OUT), lambda i: (0, 0)),
            pl.BlockSpec((1, OUT), lambda i: (0, 0)),
        ],
        out_specs=[
            pl.BlockSpec((_RB, D), lambda i: (i, 0)),
            pl.BlockSpec((G, OUT), lambda i: (0, 0)),
            pl.BlockSpec((G, OUT), lambda i: (0, 0)),
        ],
        out_shape=[
            jax.ShapeDtypeStruct((N, D), jnp.float32),
            jax.ShapeDtypeStruct((G, OUT), jnp.float32),
            jax.ShapeDtypeStruct((G, OUT), jnp.float32),
        ],
        scratch_shapes=[
            pltpu.VMEM((G, D), jnp.float32),
            pltpu.VMEM((G, D), jnp.float32),
        ],
    )(acc, h2, dinv_b, b, batch2d, wm0, bm0, wm1, bm1)


def kernel(x, edge_index, batch, W1, b1, W2, b2, W3, b3, Wm0, bm0, Wm1, bm1):
    edge3 = edge_index.reshape(2, NW, NB, KB)

    indeg = _deg_call(edge3).reshape(NC, NPAD // 128, 128)
    dinv_n = _dinv_call(indeg)

    h2 = _mm_scale(x, W1, dinv_n)
    acc = _edge_call(h2, edge3).reshape(NC, NPAD, D)
    h2 = _layer(acc, h2, dinv_n, b1.reshape(1, D), W2)
    acc = _edge_call(h2, edge3).reshape(NC, NPAD, D)
    h2 = _layer(acc, h2, dinv_n, b2.reshape(1, D), W3)
    acc = _edge_call(h2, edge3).reshape(NC, NPAD, D)

    emb, logits, probs = _head(acc, h2, dinv_n, b3.reshape(1, D),
                               batch.reshape(_NRB, _RB),
                               Wm0, bm0.reshape(1, H), Wm1, bm1.reshape(1, OUT))
    return (logits, probs, emb)
